# Initial kernel scaffold; baseline (speedup 1.0000x reference)
import jax
import jax.numpy as jnp
from jax.experimental import pallas as pl


def kernel(x, rln_g, rln_b, rW1, rb1, rW2, rb2, temp, eW1, eb1, eW2, eb2, cln_g, cln_b, cW1, cb1, cW2, cb2, oln_g, oln_b):
    raise NotImplementedError("write your pallas kernel here")


# speedup vs baseline: 1.4483x; 1.4483x over previous
import functools

import numpy as np
import jax
import jax.numpy as jnp
from jax.experimental import pallas as pl
from jax.experimental.pallas import tpu as pltpu

S, D, E, K = 2048, 768, 8, 2
HR, HE = 768, 1536
DC = 2 * D
TS = 256
NT = S // TS


def _pos_encoding():
    pos = np.arange(S)[:, None].astype(np.float32)
    div = np.exp(np.arange(0, D, 2).astype(np.float32) * (-np.log(10000.0) / D))
    pe = np.zeros((S, D), dtype=np.float32)
    pe[:, 0::2] = np.sin(pos * div)
    pe[:, 1::2] = np.cos(pos * div)
    return jnp.asarray(pe)


def _ln(x, g, b):
    m = jnp.mean(x, axis=-1, keepdims=True)
    v = jnp.mean((x - m) ** 2, axis=-1, keepdims=True)
    return (x - m) * jax.lax.rsqrt(v + 1e-5) * g + b


def _bf16_dot(a, b):
    return jnp.dot(a.astype(jnp.bfloat16), b.astype(jnp.bfloat16),
                   preferred_element_type=jnp.float32)


def _router_kernel(x_ref, pe_ref, g_ref, b_ref, w1_ref, b1_ref, w2_ref, b2_ref,
                   t_ref, xp_ref, w_ref):
    xp = x_ref[...] + pe_ref[...]
    xp_ref[...] = xp
    h = _ln(xp, g_ref[...], b_ref[...])
    a = jax.nn.gelu(_bf16_dot(h, w1_ref[...]) + b1_ref[...])
    logits = _bf16_dot(a, w2_ref[...]) + b2_ref[...]
    l = logits / t_ref[0, 0]
    m = jnp.max(l, axis=-1, keepdims=True)
    p = jnp.exp(l - m)
    probs = p / jnp.sum(p, axis=-1, keepdims=True)
    iota = jax.lax.broadcasted_iota(jnp.int32, (TS, E), 1)
    i1 = jnp.argmax(probs, axis=-1)[:, None]
    m1 = jnp.max(probs, axis=-1, keepdims=True)
    probs2 = jnp.where(iota == i1, -jnp.inf, probs)
    i2 = jnp.argmax(probs2, axis=-1)[:, None]
    m2 = jnp.max(probs2, axis=-1, keepdims=True)
    s = m1 + m2 + 1e-9
    w_ref[...] = (jnp.where(iota == i1, m1 / s, 0.0)
                  + jnp.where(iota == i2, m2 / s, 0.0))


def _expert_kernel(xp_ref, w_ref, w1_ref, b1_ref, w2_ref, b2_ref, out_ref):
    e = pl.program_id(0)
    i = pl.program_id(1)
    a = jax.nn.gelu(_bf16_dot(xp_ref[...], w1_ref[0]) + b1_ref[0])
    o = _bf16_dot(a, w2_ref[0]) + b2_ref[0]
    iota = jax.lax.broadcasted_iota(jnp.int32, (TS, E), 1)
    we = jnp.sum(jnp.where(iota == e, w_ref[...], 0.0), axis=-1, keepdims=True)
    contrib = we * o

    @pl.when(e == 0)
    def _():
        out_ref[pl.ds(i * TS, TS), :] = contrib

    @pl.when(e > 0)
    def _():
        out_ref[pl.ds(i * TS, TS), :] = out_ref[pl.ds(i * TS, TS), :] + contrib


def _combiner_kernel(c_ref, xp_ref, cg_ref, cb_ref, w1_ref, b1_ref, w2_ref,
                     b2_ref, og_ref, ob_ref, out_ref):
    ch = _ln(c_ref[...], cg_ref[...], cb_ref[...])
    a = jax.nn.gelu(_bf16_dot(ch, w1_ref[...]) + b1_ref[...])
    c = _bf16_dot(a, w2_ref[...]) + b2_ref[...]
    out_ref[...] = _ln(xp_ref[...] + c, og_ref[...], ob_ref[...])


def kernel(x, rln_g, rln_b, rW1, rb1, rW2, rb2, temp, eW1, eb1, eW2, eb2,
           cln_g, cln_b, cW1, cb1, cW2, cb2, oln_g, oln_b):
    x2 = x.reshape(S, D)
    pe = _pos_encoding()

    row = lambda v: v.reshape(1, -1)

    xp, wfull = pl.pallas_call(
        _router_kernel,
        grid=(NT,),
        in_specs=[
            pl.BlockSpec((TS, D), lambda i: (i, 0)),
            pl.BlockSpec((TS, D), lambda i: (i, 0)),
            pl.BlockSpec((1, D), lambda i: (0, 0)),
            pl.BlockSpec((1, D), lambda i: (0, 0)),
            pl.BlockSpec((D, HR), lambda i: (0, 0)),
            pl.BlockSpec((1, HR), lambda i: (0, 0)),
            pl.BlockSpec((HR, E), lambda i: (0, 0)),
            pl.BlockSpec((1, E), lambda i: (0, 0)),
            pl.BlockSpec((1, 1), lambda i: (0, 0)),
        ],
        out_specs=[
            pl.BlockSpec((TS, D), lambda i: (i, 0)),
            pl.BlockSpec((TS, E), lambda i: (i, 0)),
        ],
        out_shape=[
            jax.ShapeDtypeStruct((S, D), jnp.float32),
            jax.ShapeDtypeStruct((S, E), jnp.float32),
        ],
    )(x2, pe, row(rln_g), row(rln_b), rW1, row(rb1), rW2, row(rb2),
      temp.reshape(1, 1))

    xp_b = xp.astype(jnp.bfloat16)
    eW1_b = eW1.astype(jnp.bfloat16)
    eW2_b = eW2.astype(jnp.bfloat16)

    comb = pl.pallas_call(
        _expert_kernel,
        grid=(E, NT),
        in_specs=[
            pl.BlockSpec((TS, D), lambda e, i: (i, 0)),
            pl.BlockSpec((TS, E), lambda e, i: (i, 0)),
            pl.BlockSpec((1, D, HE), lambda e, i: (e, 0, 0)),
            pl.BlockSpec((1, 1, HE), lambda e, i: (e, 0, 0)),
            pl.BlockSpec((1, HE, D), lambda e, i: (e, 0, 0)),
            pl.BlockSpec((1, 1, D), lambda e, i: (e, 0, 0)),
        ],
        out_specs=pl.BlockSpec((S, D), lambda e, i: (0, 0)),
        out_shape=jax.ShapeDtypeStruct((S, D), jnp.float32),
    )(xp_b, wfull, eW1_b, eb1.reshape(E, 1, HE), eW2_b, eb2.reshape(E, 1, D))

    out = pl.pallas_call(
        _combiner_kernel,
        grid=(NT,),
        in_specs=[
            pl.BlockSpec((TS, D), lambda i: (i, 0)),
            pl.BlockSpec((TS, D), lambda i: (i, 0)),
            pl.BlockSpec((1, D), lambda i: (0, 0)),
            pl.BlockSpec((1, D), lambda i: (0, 0)),
            pl.BlockSpec((D, DC), lambda i: (0, 0)),
            pl.BlockSpec((1, DC), lambda i: (0, 0)),
            pl.BlockSpec((DC, D), lambda i: (0, 0)),
            pl.BlockSpec((1, D), lambda i: (0, 0)),
            pl.BlockSpec((1, D), lambda i: (0, 0)),
            pl.BlockSpec((1, D), lambda i: (0, 0)),
        ],
        out_specs=pl.BlockSpec((TS, D), lambda i: (i, 0)),
        out_shape=jax.ShapeDtypeStruct((S, D), jnp.float32),
    )(comb, xp, row(cln_g), row(cln_b), cW1, row(cb1), cW2, row(cb2),
      row(oln_g), row(oln_b))

    return out.reshape(1, S, D)
